# baseline (speedup 1.0000x reference)
import functools

import jax
import jax.numpy as jnp
from jax import lax
from jax.experimental import pallas as pl
from jax.experimental.pallas import tpu as pltpu
from jax.experimental.pallas import tpu_sc as plsc

_NC = 2
_NS = 16
_NW = _NC * _NS
_G = 128
_EPS = 1e-12


def _make_gather(n_rows, d, chj):
    per_w = n_rows // _NW
    n_chunks = per_w // chj
    mesh = plsc.VectorSubcoreMesh(core_axis_name="c", subcore_axis_name="s")

    @functools.partial(
        pl.kernel,
        mesh=mesh,
        out_type=jax.ShapeDtypeStruct((n_rows, _G, d), jnp.float32),
        scratch_types=[
            pltpu.VMEM((chj, _G), jnp.int32),
            pltpu.VMEM((chj, _G, d), jnp.float32),
            pltpu.SemaphoreType.DMA,
        ],
        compiler_params=pltpu.CompilerParams(use_tc_tiling_on_sc=False),
    )
    def gather_k(idx_hbm, table_hbm, out_hbm, idx_v, rows_v, sem):
        wid = lax.axis_index("s") * _NC + lax.axis_index("c")
        base = wid * per_w

        def body(i, carry):
            r0 = base + i * chj
            pltpu.sync_copy(idx_hbm.at[pl.ds(r0, chj)], idx_v)
            handles = [
                pltpu.async_copy(table_hbm.at[idx_v.at[j]], rows_v.at[j], sem)
                for j in range(chj)
            ]
            for h in handles:
                h.wait()
            pltpu.sync_copy(rows_v, out_hbm.at[pl.ds(r0, chj)])
            return carry

        lax.fori_loop(0, n_chunks, body, 0)

    return gather_k


def _ln_body(x_ref, pe_ref, g_ref, bta_ref, o_ref):
    x = x_ref[...] + pe_ref[...]
    mean = jnp.mean(x, axis=-1, keepdims=True)
    c = x - mean
    var = jnp.mean(c * c, axis=-1, keepdims=True)
    o_ref[...] = c * lax.rsqrt(var + _EPS) * g_ref[...] + bta_ref[...]


def _make_ln(bb, b, l, d):
    return pl.pallas_call(
        _ln_body,
        grid=(b // bb,),
        in_specs=[
            pl.BlockSpec((bb, l, d), lambda i: (i, 0, 0)),
            pl.BlockSpec((1, l, d), lambda i: (0, 0, 0)),
            pl.BlockSpec((1, 1, d), lambda i: (0, 0, 0)),
            pl.BlockSpec((1, 1, d), lambda i: (0, 0, 0)),
        ],
        out_specs=pl.BlockSpec((bb, l, d), lambda i: (i, 0, 0)),
        out_shape=jax.ShapeDtypeStruct((b, l, d), jnp.float32),
    )


def kernel(input_ids, W_emb, pe, ln_gamma, ln_beta):
    b, l = input_ids.shape
    d = W_emb.shape[1]
    n = b * l
    n_rows = n // _G
    idx = input_ids.reshape(n_rows, _G)
    gathered = _make_gather(n_rows, d, 4)(idx, W_emb)
    x = gathered.reshape(b, l, d)
    ln = _make_ln(64, b, l, d)
    return ln(x, pe[:l].reshape(1, l, d), ln_gamma.reshape(1, 1, d),
              ln_beta.reshape(1, 1, d))
